# Initial kernel scaffold; baseline (speedup 1.0000x reference)
import functools

import jax
import jax.numpy as jnp
from jax import lax
from jax.experimental import pallas as pl
from jax.experimental.pallas import tpu as pltpu
from jax.experimental.pallas import tpu_sc as plsc

B = 4096
S = 200
D = 32

NC = 2
NS = 16
NW = NC * NS
RPW = B // NW
CHUNK = 100
CPR = S // CHUNK
NCHUNK = RPW * CPR
GRP = 8
NGRP = NCHUNK // GRP


def _sc_embed_sum(words32, table):
  words_r = words32.reshape(NW, NCHUNK, CHUNK)
  mesh = plsc.VectorSubcoreMesh(core_axis_name="c", subcore_axis_name="s")

  @functools.partial(
      pl.kernel,
      out_type=jax.ShapeDtypeStruct((B, D), jnp.float32),
      mesh=mesh,
      scratch_types=[
          pltpu.VMEM((NCHUNK, CHUNK), jnp.int32),
          pltpu.VMEM((2, GRP, CHUNK, D), jnp.float32),
          pltpu.VMEM((RPW, D), jnp.float32),
          pltpu.SemaphoreType.DMA,
          pltpu.SemaphoreType.DMA,
      ],
  )
  def sc_kernel(words_hbm, table_hbm, out_hbm, idx_v, bufs, out_v, sem_a,
                sem_b):
    wid = lax.axis_index("s") * NC + lax.axis_index("c")
    pltpu.sync_copy(words_hbm.at[wid], idx_v)

    sems = (sem_a, sem_b)

    def issue(g, slot):
      for j in range(GRP):
        pltpu.async_copy(
            table_hbm.at[idx_v.at[g * GRP + j]], bufs.at[slot, j], sems[slot])

    def drain(g, slot):
      for j in range(GRP):
        pltpu.make_async_copy(
            table_hbm.at[idx_v.at[g * GRP + j]], bufs.at[slot, j],
            sems[slot]).wait()

    def process(g, slot):
      for i in range(GRP // CPR):
        z = jnp.zeros((16,), jnp.float32)

        @pl.loop(0, CHUNK, init_carry=(z, z, z, z), unroll=10)
        def acc(r, carry):
          a00, a01, a10, a11 = carry
          a00 = a00 + bufs[slot, CPR * i, r, 0:16]
          a01 = a01 + bufs[slot, CPR * i, r, 16:32]
          a10 = a10 + bufs[slot, CPR * i + 1, r, 0:16]
          a11 = a11 + bufs[slot, CPR * i + 1, r, 16:32]
          return a00, a01, a10, a11

        a00, a01, a10, a11 = acc
        row = g * (GRP // CPR) + i
        out_v[row, 0:16] = a00 + a10
        out_v[row, 16:32] = a01 + a11

    issue(0, 0)

    @pl.loop(0, NGRP, step=2)
    def _(g):
      issue(g + 1, 1)
      drain(g, 0)
      process(g, 0)

      @pl.when(g + 2 < NGRP)
      def _():
        issue(g + 2, 0)

      drain(g + 1, 1)
      process(g + 1, 1)

    pltpu.sync_copy(out_v, out_hbm.at[pl.ds(wid * RPW, RPW)])

  return sc_kernel(words_r, table)


_BB = 128


def _pos_bits_kernel(pos_ref, out_ref):
  p = pos_ref[...]
  d = lax.broadcasted_iota(jnp.int32, (1, 1, D), 2)
  bits = jnp.bitwise_and(jnp.right_shift(p[:, :, None], d), 1)
  out_ref[...] = jnp.sum(bits, axis=1).astype(jnp.float32)


def _tc_pos_sum(pos32):
  return pl.pallas_call(
      _pos_bits_kernel,
      out_shape=jax.ShapeDtypeStruct((B, D), jnp.float32),
      grid=(B // _BB,),
      in_specs=[pl.BlockSpec((_BB, S), lambda i: (i, 0))],
      out_specs=pl.BlockSpec((_BB, D), lambda i: (i, 0)),
  )(pos32)


@jax.jit
def kernel(words, positions, table):
  words32 = words.astype(jnp.int32)
  pos32 = positions.astype(jnp.int32)
  emb = _sc_embed_sum(words32, table)
  pos_sum = _tc_pos_sum(pos32)
  return emb + pos_sum


# speedup vs baseline: 14.0088x; 14.0088x over previous
import functools

import numpy as np
import jax
from jax._src.config import enable_x64 as _enable_x64
import jax.numpy as jnp
from jax import lax
from jax.experimental import pallas as pl
from jax.experimental.pallas import tpu as pltpu
from jax.experimental.pallas import tpu_sc as plsc

B = 4096
S = 200
D = 32

NC = 2
NS = 16
NW = NC * NS
RPW = B // NW
CHUNK = 100
CPR = S // CHUNK
NCHUNK = RPW * CPR
GRP = 8
NGRP = NCHUNK // GRP


def _i32(x):
  return np.int32(x) if isinstance(x, int) else x


def _sc_embed_sum(words32, table):
  words_r = words32.reshape(NW, NCHUNK, CHUNK)
  mesh = plsc.VectorSubcoreMesh(
      core_axis_name="c", subcore_axis_name="s", num_cores=NC,
      num_subcores=NS)

  @functools.partial(
      pl.kernel,
      out_type=jax.ShapeDtypeStruct((B, D), jnp.float32),
      mesh=mesh,
      scratch_types=[
          pltpu.VMEM((NCHUNK, CHUNK), jnp.int32),
          pltpu.VMEM((2, GRP, CHUNK, D), jnp.float32),
          pltpu.VMEM((RPW, D), jnp.float32),
          pltpu.SemaphoreType.DMA,
          pltpu.SemaphoreType.DMA,
      ],
      compiler_params=pltpu.CompilerParams(use_tc_tiling_on_sc=False),
  )
  def sc_kernel(words_hbm, table_hbm, out_hbm, idx_v, bufs, out_v, sem_a,
                sem_b):
    wid = lax.axis_index("s") * NC + lax.axis_index("c")
    pltpu.sync_copy(words_hbm.at[wid], idx_v)

    sems = (sem_a, sem_b)

    def issue(g, slot):
      for j in range(GRP):
        c = _i32(g * GRP + j)
        pltpu.async_copy(
            table_hbm.at[idx_v.at[c]], bufs.at[np.int32(slot), np.int32(j)],
            sems[slot])

    def drain(g, slot):
      for j in range(GRP):
        c = _i32(g * GRP + j)
        pltpu.make_async_copy(
            table_hbm.at[idx_v.at[c]], bufs.at[np.int32(slot), np.int32(j)],
            sems[slot]).wait()

    def process(g, slot):
      for i in range(GRP // CPR):
        z = jnp.zeros((16,), jnp.float32)

        sl = np.int32(slot)
        c0 = np.int32(CPR * i)
        c1 = np.int32(CPR * i + 1)

        def acc_body(r, carry):
          a00, a01, a10, a11 = carry
          a00 = a00 + bufs[sl, c0, r, 0:16]
          a01 = a01 + bufs[sl, c0, r, 16:32]
          a10 = a10 + bufs[sl, c1, r, 0:16]
          a11 = a11 + bufs[sl, c1, r, 16:32]
          return a00, a01, a10, a11

        a00, a01, a10, a11 = lax.fori_loop(
            np.int32(0), np.int32(CHUNK), acc_body, (z, z, z, z), unroll=10)
        row = _i32(g * (GRP // CPR) + i)
        out_v[row, 0:16] = a00 + a10
        out_v[row, 16:32] = a01 + a11

    issue(0, 0)

    def group_body(t, carry):
      g = t * np.int32(2)
      issue(g + 1, 1)
      drain(g, 0)
      process(g, 0)

      @pl.when(g + 2 < NGRP)
      def _():
        issue(g + 2, 0)

      drain(g + 1, 1)
      process(g + 1, 1)
      return carry

    lax.fori_loop(np.int32(0), np.int32(NGRP // 2), group_body, np.int32(0))

    pltpu.sync_copy(out_v, out_hbm.at[pl.ds(wid * RPW, RPW)])

  return sc_kernel(words_r, table)


_BB = 128


def _pos_bits_kernel(pos_ref, out_ref):
  p = pos_ref[...]
  d = lax.broadcasted_iota(jnp.int32, (1, 1, D), 2)
  bits = jnp.bitwise_and(jnp.right_shift(p[:, :, None], d), 1)
  out_ref[...] = jnp.sum(bits, axis=1).astype(jnp.float32)


def _tc_pos_sum(pos32):
  return pl.pallas_call(
      _pos_bits_kernel,
      out_shape=jax.ShapeDtypeStruct((B, D), jnp.float32),
      grid=(B // _BB,),
      in_specs=[pl.BlockSpec((_BB, S), lambda i: (i, 0))],
      out_specs=pl.BlockSpec((_BB, D), lambda i: (i, 0)),
  )(pos32)


@jax.jit
def kernel(words, positions, table):
  with _enable_x64(False):
    words32 = words.astype(jnp.int32)
    pos32 = positions.astype(jnp.int32)
    emb = _sc_embed_sum(words32, table)
    pos_sum = _tc_pos_sum(pos32)
    return emb + pos_sum


# speedup vs baseline: 21.7451x; 1.5522x over previous
import functools

import numpy as np
import jax
from jax._src.config import enable_x64 as _enable_x64
import jax.numpy as jnp
from jax import lax
from jax.experimental import pallas as pl
from jax.experimental.pallas import tpu as pltpu
from jax.experimental.pallas import tpu_sc as plsc

B = 4096
S = 200
D = 32

NC = 2
NS = 16
NW = NC * NS
RPW = B // NW
CHUNK = 100
CPR = S // CHUNK
NCHUNK = RPW * CPR
GRP = 8
NGRP = NCHUNK // GRP


def _i32(x):
  return np.int32(x) if isinstance(x, int) else x


def _sc_embed_sum(words32, table):
  words_r = words32.reshape(NW, NCHUNK, CHUNK)
  mesh = plsc.VectorSubcoreMesh(
      core_axis_name="c", subcore_axis_name="s", num_cores=NC,
      num_subcores=NS)

  @functools.partial(
      pl.kernel,
      out_type=jax.ShapeDtypeStruct((B, D), jnp.float32),
      mesh=mesh,
      scratch_types=[
          pltpu.VMEM((NCHUNK, CHUNK), jnp.int32),
          pltpu.VMEM((2, GRP, CHUNK, D), jnp.float32),
          pltpu.VMEM((RPW, D), jnp.float32),
          pltpu.SemaphoreType.DMA,
          pltpu.SemaphoreType.DMA,
      ],
      compiler_params=pltpu.CompilerParams(use_tc_tiling_on_sc=False),
  )
  def sc_kernel(words_hbm, table_hbm, out_hbm, idx_v, bufs, out_v, sem_a,
                sem_b):
    wid = lax.axis_index("s") * NC + lax.axis_index("c")
    pltpu.sync_copy(words_hbm.at[wid], idx_v)

    sems = (sem_a, sem_b)

    def issue(g, slot):
      for j in range(GRP):
        c = _i32(g * GRP + j)
        pltpu.async_copy(
            table_hbm.at[idx_v.at[c]], bufs.at[np.int32(slot), np.int32(j)],
            sems[slot])

    def drain(g, slot):
      for j in range(GRP):
        c = _i32(g * GRP + j)
        pltpu.make_async_copy(
            table_hbm.at[idx_v.at[c]], bufs.at[np.int32(slot), np.int32(j)],
            sems[slot]).wait()

    def process(g, slot):
      for i in range(GRP // CPR):
        z = jnp.zeros((16,), jnp.float32)

        sl = np.int32(slot)
        c0 = np.int32(CPR * i)
        c1 = np.int32(CPR * i + 1)

        def acc_body(r, carry):
          a00, a01, a10, a11 = carry
          a00 = a00 + bufs[sl, c0, r, 0:16]
          a01 = a01 + bufs[sl, c0, r, 16:32]
          a10 = a10 + bufs[sl, c1, r, 0:16]
          a11 = a11 + bufs[sl, c1, r, 16:32]
          return a00, a01, a10, a11

        a00, a01, a10, a11 = lax.fori_loop(
            np.int32(0), np.int32(CHUNK), acc_body, (z, z, z, z), unroll=10)
        row = _i32(g * (GRP // CPR) + i)
        out_v[row, 0:16] = a00 + a10
        out_v[row, 16:32] = a01 + a11

    issue(0, 0)

    def group_body(t, carry):
      g = t * np.int32(2)
      issue(g + 1, 1)
      drain(g, 0)
      process(g, 0)

      @pl.when(g + 2 < NGRP)
      def _():
        issue(g + 2, 0)

      drain(g + 1, 1)
      process(g + 1, 1)
      return carry

    lax.fori_loop(np.int32(0), np.int32(NGRP // 2), group_body, np.int32(0))

    pltpu.sync_copy(out_v, out_hbm.at[pl.ds(wid * RPW, RPW)])

  return sc_kernel(words_r, table)


_BB = 512


def _pos_bits_kernel(pos_ref, out_ref):
  p = pos_ref[...]
  p = jnp.concatenate(
      [p, jnp.zeros((_BB, 256 - S), jnp.int32)], axis=1)

  planes = [p]
  w = 256
  while w > 1:
    h = w // 2
    nxt = []
    carry = None
    for pj in planes:
      lo = pj[:, :h]
      hi = pj[:, h:]
      if carry is None:
        s = lo ^ hi
        c = lo & hi
      else:
        x = lo ^ hi
        s = x ^ carry
        c = (lo & hi) | (carry & x)
      nxt.append(s)
      carry = c
    nxt.append(carry)
    planes = nxt
    w = h

  d_iota = lax.broadcasted_iota(jnp.int32, (1, D), 1)
  acc = jnp.zeros((_BB, D), jnp.int32)
  for j, pj in enumerate(planes):
    acc = acc + (((pj >> d_iota) & 1) << j)
  out_ref[...] = acc.astype(jnp.float32)


def _tc_pos_sum(pos32):
  return pl.pallas_call(
      _pos_bits_kernel,
      out_shape=jax.ShapeDtypeStruct((B, D), jnp.float32),
      grid=(B // _BB,),
      in_specs=[pl.BlockSpec((_BB, S), lambda i: (i, 0))],
      out_specs=pl.BlockSpec((_BB, D), lambda i: (i, 0)),
  )(pos32)


@jax.jit
def kernel(words, positions, table):
  with _enable_x64(False):
    words32 = words.astype(jnp.int32)
    pos32 = positions.astype(jnp.int32)
    emb = _sc_embed_sum(words32, table)
    pos_sum = _tc_pos_sum(pos32)
    return emb + pos_sum
